# baseline (speedup 1.0000x reference)
import functools

import jax
import jax.numpy as jnp
from jax import lax
from jax.experimental import pallas as pl
from jax.experimental.pallas import tpu as pltpu
from jax.experimental.pallas import tpu_sc as plsc

_NUM_CORES = 2
_NUM_SUBCORES = 16
_NUM_WORKERS = _NUM_CORES * _NUM_SUBCORES
_LANES = 16


def _top2(logits, e):
    tt = logits.shape[0]
    iota = jax.lax.broadcasted_iota(jnp.int32, (tt, e), 1)
    m1 = jnp.max(logits, axis=1, keepdims=True)
    idx1 = jnp.min(jnp.where(logits == m1, iota, e), axis=1, keepdims=True)
    masked = jnp.where(iota == idx1, -jnp.inf, logits)
    m2 = jnp.max(masked, axis=1, keepdims=True)
    idx2 = jnp.min(jnp.where(masked == m2, iota, e), axis=1, keepdims=True)
    return iota, m1, idx1, m2, idx2


def _fused_body(x_ref, w_ref, probs_ref, map_ref):
    ts, b, d = x_ref.shape
    e = w_ref.shape[0]
    x = x_ref[...].reshape(ts * b, d)
    logits = jax.lax.dot_general(
        x, w_ref[...], (((1,), (1,)), ((), ())),
        preferred_element_type=jnp.float32,
    )
    iota, m1, idx1, m2, idx2 = _top2(logits, e)
    t = jnp.exp(m2 - m1)
    denom = 1.0 + t
    p1 = 1.0 / denom
    p2 = t / denom
    probs = jnp.where(iota == idx1, p1, jnp.where(iota == idx2, p2, 0.0))
    rmap = (iota == idx1) | (iota == idx2)
    probs_ref[...] = probs
    map_ref[...] = rmap


@functools.partial(jax.jit, static_argnames=("ts",))
def _route_fused3d(h, w, ts):
    s, b, d = h.shape
    e = w.shape[0]
    return pl.pallas_call(
        _fused_body,
        grid=(s // ts,),
        in_specs=[
            pl.BlockSpec((ts, b, d), lambda i: (i, 0, 0)),
            pl.BlockSpec((e, d), lambda i: (0, 0)),
        ],
        out_specs=[
            pl.BlockSpec((ts * b, e), lambda i: (i, 0)),
            pl.BlockSpec((ts * b, e), lambda i: (i, 0)),
        ],
        out_shape=[
            jax.ShapeDtypeStruct((s * b, e), jnp.float32),
            jax.ShapeDtypeStruct((s * b, e), jnp.bool_),
        ],
        compiler_params=pltpu.CompilerParams(
            dimension_semantics=("arbitrary",),
        ),
    )(h, w)


def kernel(hidden_states, router_weight):
    return _route_fused3d(
        hidden_states.astype(jnp.float32),
        router_weight.astype(jnp.float32),
        ts=256,
    )
